# baseline (speedup 1.0000x reference)
import jax
import jax.numpy as jnp
from jax.experimental import pallas as pl
from jax.experimental.pallas import tpu as pltpu


def _scatter_body(idx_ref, mem_any, val_blk, out_blk):
    del idx_ref, mem_any
    out_blk[...] = val_blk[...]


def kernel(mem, idx, val):
    M, D = mem.shape
    B, _ = val.shape
    mem3 = mem.reshape(M, 1, D)
    val3 = val.reshape(B, 1, D)
    out = pl.pallas_call(
        _scatter_body,
        grid_spec=pltpu.PrefetchScalarGridSpec(
            num_scalar_prefetch=1,
            grid=(B,),
            in_specs=[
                pl.BlockSpec(memory_space=pltpu.MemorySpace.HBM),
                pl.BlockSpec((1, 1, D), lambda i, idx_ref: (i, 0, 0)),
            ],
            out_specs=pl.BlockSpec((1, 1, D), lambda i, idx_ref: (idx_ref[i], 0, 0)),
        ),
        out_shape=jax.ShapeDtypeStruct((M, 1, D), mem.dtype),
        input_output_aliases={1: 0},
    )(idx, mem3, val3)
    return out.reshape(M, D)


# speedup vs baseline: 2.0943x; 2.0943x over previous
import functools

import jax
import jax.numpy as jnp
from jax import lax
from jax.experimental import pallas as pl
from jax.experimental.pallas import tpu as pltpu
from jax.experimental.pallas import tpu_sc as plsc

NC = 2
NS = 16
NW = NC * NS
L = 16

CHUNK = 128


def _sc_body(M, B, R, mem_hbm, idx_hbm, val_hbm, out_hbm,
             idx_v, row_buf, pos_buf, table, rot16, keep16, rows_v,
             sem_copy, sem_g, sem_s):
    tail = M - (NW - 1) * R
    wid = lax.axis_index("s") * NC + lax.axis_index("c")
    lo = wid * R
    hi = jnp.minimum(lo + R, M)
    iota = lax.iota(jnp.int32, L)

    @pl.when(wid < NW - 1)
    def _():
        pltpu.make_async_copy(
            mem_hbm.at[pl.ds(lo, R)], out_hbm.at[pl.ds(lo, R)],
            sem_copy).start()

    @pl.when(wid == NW - 1)
    def _():
        pltpu.make_async_copy(
            mem_hbm.at[pl.ds(lo, tail)], out_hbm.at[pl.ds(lo, tail)],
            sem_copy).start()

    pltpu.sync_copy(idx_hbm, idx_v)

    def scan_body(i, cursor):
        x = plsc.load_gather(idx_v, [i * L + iota])
        pos = i * L + iota
        m = (x >= lo) & (x < hi)
        mi = m.astype(jnp.int32)
        cnt = jnp.sum(mi)

        @pl.when(cnt > 0)
        def _():
            dest = cursor + jnp.cumsum(mi) - 1
            plsc.store_scatter(
                row_buf, [dest >> 7, dest & (CHUNK - 1)], x, mask=m)
            plsc.store_scatter(
                pos_buf, [dest >> 7, dest & (CHUNK - 1)], pos, mask=m)
            keep16[...] = mi

            @pl.when(cnt > 1)
            def _():
                rot16[...] = jnp.where(m, x, -1)
                dup = jnp.zeros((L,), jnp.bool_)
                for r in range(1, L):
                    y = plsc.load_gather(rot16, [(iota + r) & (L - 1)])
                    later = (iota + r) < L
                    dup = dup | (m & later & (y == x))
                keep16[...] = jnp.where(dup, 0, mi)

            keep = keep16[...] > 0
            plsc.store_scatter(table, [x - lo], pos, mask=keep)

        return cursor + cnt

    n_cand = lax.fori_loop(0, B // L, scan_body, jnp.int32(0))

    def win_body(c, carry):
        wcur, pad_row, pad_pos = carry
        fl = c * L + iota
        valid = fl < n_cand
        fb, fc = fl >> 7, fl & (CHUNK - 1)
        x = plsc.load_gather(row_buf, [fb, fc], mask=valid)
        p = plsc.load_gather(pos_buf, [fb, fc], mask=valid)
        w = plsc.load_gather(table, [jnp.where(valid, x - lo, 0)], mask=valid)
        keep = valid & (w == p)
        ki = keep.astype(jnp.int32)
        kcnt = jnp.sum(ki)
        dest = wcur + jnp.cumsum(ki) - 1
        plsc.store_scatter(row_buf, [dest >> 7, dest & (CHUNK - 1)], x, mask=keep)
        plsc.store_scatter(pos_buf, [dest >> 7, dest & (CHUNK - 1)], p, mask=keep)
        lmax = jnp.max(jnp.where(keep, iota, -1))
        sel = keep & (iota == lmax)
        pr = jnp.max(jnp.where(sel, x, -1))
        pp = jnp.max(jnp.where(sel, p, -1))
        pad_row = jnp.where(kcnt > 0, pr, pad_row)
        pad_pos = jnp.where(kcnt > 0, pp, pad_pos)
        return wcur + kcnt, pad_row, pad_pos

    n_win, pad_row, pad_pos = lax.fori_loop(
        0, pl.cdiv(n_cand, L), win_body,
        (jnp.int32(0), jnp.int32(0), jnp.int32(0)))

    n_tot = pl.cdiv(n_win, CHUNK) * CHUNK

    def pad_body(c, _):
        e = n_win + c * L + iota
        mm = e < n_tot
        plsc.store_scatter(
            row_buf, [e >> 7, e & (CHUNK - 1)],
            jnp.full((L,), pad_row, jnp.int32), mask=mm)
        plsc.store_scatter(
            pos_buf, [e >> 7, e & (CHUNK - 1)],
            jnp.full((L,), pad_pos, jnp.int32), mask=mm)
        return 0

    lax.fori_loop(0, pl.cdiv(n_tot - n_win, L), pad_body, 0)

    @pl.when(wid < NW - 1)
    def _():
        pltpu.make_async_copy(
            mem_hbm.at[pl.ds(lo, R)], out_hbm.at[pl.ds(lo, R)],
            sem_copy).wait()

    @pl.when(wid == NW - 1)
    def _():
        pltpu.make_async_copy(
            mem_hbm.at[pl.ds(lo, tail)], out_hbm.at[pl.ds(lo, tail)],
            sem_copy).wait()

    def gs_body(b, _):
        g = pltpu.make_async_copy(val_hbm.at[pos_buf.at[b]], rows_v, sem_g)
        g.start()
        g.wait()
        s = pltpu.make_async_copy(rows_v, out_hbm.at[row_buf.at[b]], sem_s)
        s.start()
        s.wait()
        return 0

    lax.fori_loop(0, n_tot // CHUNK, gs_body, 0)


def kernel(mem, idx, val):
    M, D = mem.shape
    B, _ = val.shape
    assert B % L == 0
    R = ((M + NW - 1) // NW + 7) // 8 * 8
    assert 0 < M - (NW - 1) * R <= R

    mesh = plsc.VectorSubcoreMesh(
        core_axis_name="c", subcore_axis_name="s", num_cores=NC)

    sc = pl.kernel(
        functools.partial(_sc_body, M, B, R),
        out_type=jax.ShapeDtypeStruct((M, D), jnp.float32),
        mesh=mesh,
        compiler_params=pltpu.CompilerParams(needs_layout_passes=False),
        scratch_types=[
            pltpu.VMEM((B,), jnp.int32),
            pltpu.VMEM((B // CHUNK, CHUNK), jnp.int32),
            pltpu.VMEM((B // CHUNK, CHUNK), jnp.int32),
            pltpu.VMEM((R,), jnp.int32),
            pltpu.VMEM((L,), jnp.int32),
            pltpu.VMEM((L,), jnp.int32),
            pltpu.VMEM((CHUNK, D), jnp.float32),
            pltpu.SemaphoreType.DMA,
            pltpu.SemaphoreType.DMA,
            pltpu.SemaphoreType.DMA,
        ],
    )
    return sc(mem, idx, val)


# speedup vs baseline: 45.9234x; 21.9277x over previous
import functools

import jax
import jax.numpy as jnp
from jax import lax
from jax.experimental import pallas as pl
from jax.experimental.pallas import tpu as pltpu
from jax.experimental.pallas import tpu_sc as plsc

NC = 2
NS = 16
NW = NC * NS
L = 16

CHUNK = 128


def _sc_body(M, B, R, idx_hbm, val_hbm, out_hbm,
             idx_v, row_buf, pos_buf, table, rot16, keep16, rows_v,
             sem_g, sem_s):
    wid = lax.axis_index("s") * NC + lax.axis_index("c")
    lo = wid * R
    hi = jnp.minimum(lo + R, M)
    iota = lax.iota(jnp.int32, L)

    pltpu.sync_copy(idx_hbm, idx_v)

    def scan_body(i, cursor):
        x = plsc.load_gather(idx_v, [i * L + iota])
        pos = i * L + iota
        m = (x >= lo) & (x < hi)
        mi = m.astype(jnp.int32)
        cnt = jnp.sum(mi)

        @pl.when(cnt > 0)
        def _():
            dest = cursor + jnp.cumsum(mi) - 1
            plsc.store_scatter(
                row_buf, [dest >> 7, dest & (CHUNK - 1)], x, mask=m)
            plsc.store_scatter(
                pos_buf, [dest >> 7, dest & (CHUNK - 1)], pos, mask=m)
            keep16[...] = mi

            @pl.when(cnt > 1)
            def _():
                rot16[...] = jnp.where(m, x, -1)
                dup = jnp.zeros((L,), jnp.bool_)
                for r in range(1, L):
                    y = plsc.load_gather(rot16, [(iota + r) & (L - 1)])
                    later = (iota + r) < L
                    dup = dup | (m & later & (y == x))
                keep16[...] = jnp.where(dup, 0, mi)

            keep = keep16[...] > 0
            plsc.store_scatter(table, [x - lo], pos, mask=keep)

        return cursor + cnt

    n_cand = lax.fori_loop(0, B // L, scan_body, jnp.int32(0))

    def win_body(c, carry):
        wcur, pad_row, pad_pos = carry
        fl = c * L + iota
        valid = fl < n_cand
        fb, fc = fl >> 7, fl & (CHUNK - 1)
        x = plsc.load_gather(row_buf, [fb, fc], mask=valid)
        p = plsc.load_gather(pos_buf, [fb, fc], mask=valid)
        w = plsc.load_gather(table, [jnp.where(valid, x - lo, 0)], mask=valid)
        keep = valid & (w == p)
        ki = keep.astype(jnp.int32)
        kcnt = jnp.sum(ki)
        dest = wcur + jnp.cumsum(ki) - 1
        plsc.store_scatter(row_buf, [dest >> 7, dest & (CHUNK - 1)], x, mask=keep)
        plsc.store_scatter(pos_buf, [dest >> 7, dest & (CHUNK - 1)], p, mask=keep)
        lmax = jnp.max(jnp.where(keep, iota, -1))
        sel = keep & (iota == lmax)
        pr = jnp.max(jnp.where(sel, x, -1))
        pp = jnp.max(jnp.where(sel, p, -1))
        pad_row = jnp.where(kcnt > 0, pr, pad_row)
        pad_pos = jnp.where(kcnt > 0, pp, pad_pos)
        return wcur + kcnt, pad_row, pad_pos

    n_win, pad_row, pad_pos = lax.fori_loop(
        0, pl.cdiv(n_cand, L), win_body,
        (jnp.int32(0), jnp.int32(0), jnp.int32(0)))

    n_tot = pl.cdiv(n_win, CHUNK) * CHUNK

    def pad_body(c, _):
        e = n_win + c * L + iota
        mm = e < n_tot
        plsc.store_scatter(
            row_buf, [e >> 7, e & (CHUNK - 1)],
            jnp.full((L,), pad_row, jnp.int32), mask=mm)
        plsc.store_scatter(
            pos_buf, [e >> 7, e & (CHUNK - 1)],
            jnp.full((L,), pad_pos, jnp.int32), mask=mm)
        return 0

    lax.fori_loop(0, pl.cdiv(n_tot - n_win, L), pad_body, 0)

    def gs_body(b, _):
        g = pltpu.make_async_copy(val_hbm.at[pos_buf.at[b]], rows_v, sem_g)
        g.start()
        g.wait()
        s = pltpu.make_async_copy(rows_v, out_hbm.at[row_buf.at[b]], sem_s)
        s.start()
        s.wait()
        return 0

    lax.fori_loop(0, n_tot // CHUNK, gs_body, 0)


def kernel(mem, idx, val):
    M, D = mem.shape
    B, _ = val.shape
    assert B % L == 0
    R = (M + NW - 1) // NW
    assert 0 < M - (NW - 1) * R <= R

    mesh = plsc.VectorSubcoreMesh(
        core_axis_name="c", subcore_axis_name="s", num_cores=NC)

    sc = pl.kernel(
        functools.partial(_sc_body, M, B, R),
        out_type=(),
        mesh=mesh,
        compiler_params=pltpu.CompilerParams(needs_layout_passes=False),
        scratch_types=[
            pltpu.VMEM((B,), jnp.int32),
            pltpu.VMEM((B // CHUNK, CHUNK), jnp.int32),
            pltpu.VMEM((B // CHUNK, CHUNK), jnp.int32),
            pltpu.VMEM((R,), jnp.int32),
            pltpu.VMEM((L,), jnp.int32),
            pltpu.VMEM((L,), jnp.int32),
            pltpu.VMEM((CHUNK, D), jnp.float32),
            pltpu.SemaphoreType.DMA,
            pltpu.SemaphoreType.DMA,
        ],
    )

    out_ref = jax.new_ref(mem)
    sc(idx, val, out_ref)
    return out_ref[...]
